# Initial kernel scaffold; baseline (speedup 1.0000x reference)
import jax
import jax.numpy as jnp
from jax.experimental import pallas as pl


def kernel(x, pe):
    raise NotImplementedError("write your pallas kernel here")


# speedup vs baseline: 3.9511x; 3.9511x over previous
import functools

import jax
import jax.numpy as jnp
from jax import lax
from jax.experimental import pallas as pl
from jax.experimental.pallas import tpu as pltpu
from jax.experimental.pallas import tpu_sc as plsc

D_MODEL = 128
MAX_LEN = 366
NUM_CORES = 2
NUM_SUBCORES = 16
NUM_WORKERS = NUM_CORES * NUM_SUBCORES
GROUP = 128


def _body(n_groups, pe_hbm, x_hbm, out_hbm, idx_v, rows_v, sem):
    wid = lax.axis_index("s") * NUM_CORES + lax.axis_index("c")
    row0 = wid * n_groups

    pltpu.sync_copy(x_hbm.at[pl.ds(row0, n_groups)], idx_v)

    def clip_body(i, _):
        r = i // (GROUP // 16)
        c = (i % (GROUP // 16)) * 16
        v = idx_v[r, pl.ds(c, 16)]
        idx_v[r, pl.ds(c, 16)] = lax.max(lax.min(v, MAX_LEN), 1) - 1
        return 0

    lax.fori_loop(0, n_groups * (GROUP // 16), clip_body, 0)

    def gather_body(g, _):
        pltpu.async_copy(pe_hbm.at[idx_v.at[g]], rows_v, sem).wait()
        pltpu.sync_copy(rows_v, out_hbm.at[pl.ds((row0 + g) * GROUP, GROUP)])
        return 0

    lax.fori_loop(0, n_groups, gather_body, 0)


@functools.partial(jax.jit, static_argnames=())
def kernel(x, pe):
    b, h = x.shape
    n = b * h
    assert n % (NUM_WORKERS * GROUP) == 0
    n_groups = n // (NUM_WORKERS * GROUP)
    x2d = x.reshape(n // GROUP, GROUP)

    mesh = plsc.VectorSubcoreMesh(core_axis_name="c", subcore_axis_name="s")
    run = pl.kernel(
        functools.partial(_body, n_groups),
        mesh=mesh,
        out_type=jax.ShapeDtypeStruct((n, D_MODEL), jnp.float32),
        scratch_types=[
            pltpu.VMEM((n_groups, GROUP), jnp.int32),
            pltpu.VMEM((GROUP, D_MODEL), jnp.float32),
            pltpu.SemaphoreType.DMA,
        ],
    )
    out = run(pe, x2d)
    return out.reshape(b, h, D_MODEL)


# speedup vs baseline: 4.0176x; 1.0168x over previous
import functools

import jax
import jax.numpy as jnp
from jax import lax
from jax.experimental import pallas as pl
from jax.experimental.pallas import tpu as pltpu
from jax.experimental.pallas import tpu_sc as plsc

D_MODEL = 128
MAX_LEN = 366
NUM_CORES = 2
NUM_SUBCORES = 16
NUM_WORKERS = NUM_CORES * NUM_SUBCORES
GROUP = 128
NBUF = 4


def _body(n_groups, pe_hbm, x_hbm, out_hbm, idx_v, bufs, gsems, ssems):
    wid = lax.axis_index("s") * NUM_CORES + lax.axis_index("c")
    row0 = wid * n_groups

    pltpu.sync_copy(x_hbm.at[pl.ds(row0, n_groups)], idx_v)

    def clip_group(g):
        for c in range(0, GROUP, 16):
            v = idx_v[g, pl.ds(c, 16)]
            idx_v[g, pl.ds(c, 16)] = lax.max(lax.min(v, MAX_LEN), 1) - 1

    for b in range(NBUF):
        clip_group(b)
        pltpu.async_copy(pe_hbm.at[idx_v.at[b]], bufs[b], gsems[b])

    def ring_body(i, _):
        g0 = i * NBUF
        for b in range(NBUF):
            g = g0 + b
            gn = g + NBUF
            pltpu.make_async_copy(pe_hbm.at[idx_v.at[g]], bufs[b], gsems[b]).wait()
            scat = pltpu.async_copy(
                bufs[b], out_hbm.at[pl.ds((row0 + g) * GROUP, GROUP)], ssems[b]
            )

            @pl.when(gn < n_groups)
            def _():
                clip_group(gn)
                scat.wait()
                pltpu.async_copy(pe_hbm.at[idx_v.at[gn]], bufs[b], gsems[b])

            @pl.when(gn >= n_groups)
            def _():
                scat.wait()

        return 0

    lax.fori_loop(0, n_groups // NBUF, ring_body, 0)


@functools.partial(jax.jit, static_argnames=())
def kernel(x, pe):
    b, h = x.shape
    n = b * h
    assert n % (NUM_WORKERS * GROUP) == 0
    n_groups = n // (NUM_WORKERS * GROUP)
    assert n_groups % NBUF == 0
    x2d = x.reshape(n // GROUP, GROUP)

    mesh = plsc.VectorSubcoreMesh(core_axis_name="c", subcore_axis_name="s")
    run = pl.kernel(
        functools.partial(_body, n_groups),
        mesh=mesh,
        out_type=jax.ShapeDtypeStruct((n, D_MODEL), jnp.float32),
        scratch_types=[
            pltpu.VMEM((n_groups, GROUP), jnp.int32),
            [pltpu.VMEM((GROUP, D_MODEL), jnp.float32) for _ in range(NBUF)],
            [pltpu.SemaphoreType.DMA for _ in range(NBUF)],
            [pltpu.SemaphoreType.DMA for _ in range(NBUF)],
        ],
    )
    out = run(pe, x2d)
    return out.reshape(b, h, D_MODEL)


# speedup vs baseline: 15.7236x; 3.9136x over previous
import functools

import jax
import jax.numpy as jnp
from jax import lax
from jax.experimental import pallas as pl
from jax.experimental.pallas import tpu as pltpu
from jax.experimental.pallas import tpu_sc as plsc

D_MODEL = 128
MAX_LEN = 366
NUM_CORES = 2
NUM_SUBCORES = 16
NUM_WORKERS = NUM_CORES * NUM_SUBCORES
GROUP = 128
WINDOW = 8


def _body(n_groups, pe_hbm, x_hbm, out_hbm, table_v, idx_v, bufs, sem, ssems):
    wid = lax.axis_index("s") * NUM_CORES + lax.axis_index("c")
    row0 = wid * n_groups

    @pl.when(lax.axis_index("s") == 0)
    def _():
        pltpu.sync_copy(pe_hbm, table_v)

    plsc.subcore_barrier()
    pltpu.sync_copy(x_hbm.at[pl.ds(row0, n_groups)], idx_v)

    def clip_group(g):
        for c in range(0, GROUP, 16):
            v = idx_v[g, pl.ds(c, 16)]
            idx_v[g, pl.ds(c, 16)] = lax.max(lax.min(v, MAX_LEN), 1) - 1

    for b in range(2):
        clip_group(b)
        pltpu.async_copy(table_v.at[idx_v.at[b]], bufs[b], sem)

    def ring_body(g0, _):
        for b in range(2):
            g = g0 * 2 + b
            gn = g + 2
            pltpu.make_async_copy(table_v.at[idx_v.at[g]], bufs[b], sem).wait()
            scat = pltpu.async_copy(
                bufs[b], out_hbm.at[pl.ds((row0 + g) * GROUP, GROUP)], ssems[b]
            )

            @pl.when(gn < n_groups)
            def _():
                clip_group(gn)
                scat.wait()
                pltpu.async_copy(table_v.at[idx_v.at[gn]], bufs[b], sem)

            @pl.when(gn >= n_groups)
            def _():
                scat.wait()

        return 0

    lax.fori_loop(0, n_groups // 2, ring_body, 0)


@functools.partial(jax.jit, static_argnames=())
def kernel(x, pe):
    b, h = x.shape
    n = b * h
    assert n % (NUM_WORKERS * GROUP) == 0
    n_groups = n // (NUM_WORKERS * GROUP)
    x2d = x.reshape(n // GROUP, GROUP)

    mesh = plsc.VectorSubcoreMesh(core_axis_name="c", subcore_axis_name="s")
    run = pl.kernel(
        functools.partial(_body, n_groups),
        mesh=mesh,
        out_type=jax.ShapeDtypeStruct((n, D_MODEL), jnp.float32),
        scratch_types=[
            pltpu.VMEM_SHARED((MAX_LEN, D_MODEL), jnp.float32),
            pltpu.VMEM((n_groups, GROUP), jnp.int32),
            [pltpu.VMEM((GROUP, D_MODEL), jnp.float32) for _ in range(2)],
            pltpu.SemaphoreType.DMA,
            [pltpu.SemaphoreType.DMA for _ in range(2)],
        ],
    )
    out = run(pe, x2d)
    return out.reshape(b, h, D_MODEL)


# speedup vs baseline: 16.0318x; 1.0196x over previous
import functools

import jax
import jax.numpy as jnp
from jax import lax
from jax.experimental import pallas as pl
from jax.experimental.pallas import tpu as pltpu
from jax.experimental.pallas import tpu_sc as plsc

D_MODEL = 128
MAX_LEN = 366
NUM_CORES = 2
NUM_SUBCORES = 16
NUM_WORKERS = NUM_CORES * NUM_SUBCORES
GROUP = 128
NBUF = 4


def _body(n_groups, pe_hbm, x_hbm, out_hbm, table_v, idx_v, bufs, sem, ssems):
    wid = lax.axis_index("s") * NUM_CORES + lax.axis_index("c")
    row0 = wid * n_groups

    @pl.when(lax.axis_index("s") == 0)
    def _():
        pltpu.sync_copy(pe_hbm, table_v)

    plsc.subcore_barrier()
    pltpu.sync_copy(x_hbm.at[pl.ds(row0, n_groups)], idx_v)

    def clip_group(g):
        for c in range(0, GROUP, 16):
            v = idx_v[g, pl.ds(c, 16)]
            idx_v[g, pl.ds(c, 16)] = lax.max(lax.min(v, MAX_LEN), 1) - 1

    for b in range(NBUF):
        clip_group(b)
        pltpu.async_copy(table_v.at[idx_v.at[b]], bufs[b], sem)

    def ring_body(g0, _):
        for b in range(NBUF):
            g = g0 * NBUF + b
            gn = g + NBUF
            pltpu.make_async_copy(table_v.at[idx_v.at[g]], bufs[b], sem).wait()
            scat = pltpu.async_copy(
                bufs[b], out_hbm.at[pl.ds((row0 + g) * GROUP, GROUP)], ssems[b]
            )

            @pl.when(gn < n_groups)
            def _():
                clip_group(gn)
                scat.wait()
                pltpu.async_copy(table_v.at[idx_v.at[gn]], bufs[b], sem)

            @pl.when(gn >= n_groups)
            def _():
                scat.wait()

        return 0

    lax.fori_loop(0, n_groups // NBUF, ring_body, 0)


@functools.partial(jax.jit, static_argnames=())
def kernel(x, pe):
    b, h = x.shape
    n = b * h
    assert n % (NUM_WORKERS * GROUP) == 0
    n_groups = n // (NUM_WORKERS * GROUP)
    x2d = x.reshape(n // GROUP, GROUP)

    mesh = plsc.VectorSubcoreMesh(core_axis_name="c", subcore_axis_name="s")
    run = pl.kernel(
        functools.partial(_body, n_groups),
        mesh=mesh,
        out_type=jax.ShapeDtypeStruct((n, D_MODEL), jnp.float32),
        scratch_types=[
            pltpu.VMEM_SHARED((MAX_LEN, D_MODEL), jnp.float32),
            pltpu.VMEM((n_groups, GROUP), jnp.int32),
            [pltpu.VMEM((GROUP, D_MODEL), jnp.float32) for _ in range(NBUF)],
            pltpu.SemaphoreType.DMA,
            [pltpu.SemaphoreType.DMA for _ in range(NBUF)],
        ],
    )
    out = run(pe, x2d)
    return out.reshape(b, h, D_MODEL)


# speedup vs baseline: 16.1188x; 1.0054x over previous
import functools

import jax
import jax.numpy as jnp
from jax import lax
from jax.experimental import pallas as pl
from jax.experimental.pallas import tpu as pltpu
from jax.experimental.pallas import tpu_sc as plsc

D_MODEL = 128
MAX_LEN = 366
NUM_CORES = 2
NUM_SUBCORES = 16
NUM_WORKERS = NUM_CORES * NUM_SUBCORES
GROUP = 128
NBUF = 5


def _body(n_groups, pe_hbm, x_hbm, out_hbm, table_v, idx_v, bufs, sem, ssems):
    wid = lax.axis_index("s") * NUM_CORES + lax.axis_index("c")
    row0 = wid * n_groups

    @pl.when(lax.axis_index("s") == 0)
    def _():
        pltpu.sync_copy(pe_hbm, table_v)

    plsc.subcore_barrier()
    pltpu.sync_copy(x_hbm.at[pl.ds(row0, n_groups)], idx_v)

    def clip_group(g):
        for c in range(0, GROUP, 16):
            v = idx_v[g, pl.ds(c, 16)]
            idx_v[g, pl.ds(c, 16)] = lax.max(lax.min(v, MAX_LEN), 1) - 1

    for b in range(NBUF):
        clip_group(b)
        pltpu.async_copy(table_v.at[idx_v.at[b]], bufs[b], sem)

    def ring_body(g0, _):
        for b in range(NBUF):
            g = g0 * NBUF + b
            gn = g + NBUF
            pltpu.make_async_copy(table_v.at[idx_v.at[g]], bufs[b], sem).wait()
            scat = pltpu.async_copy(
                bufs[b], out_hbm.at[pl.ds((row0 + g) * GROUP, GROUP)], ssems[b]
            )

            @pl.when(gn < n_groups)
            def _():
                clip_group(gn)
                scat.wait()
                pltpu.async_copy(table_v.at[idx_v.at[gn]], bufs[b], sem)

            @pl.when(gn >= n_groups)
            def _():
                scat.wait()

        return 0

    lax.fori_loop(0, n_groups // NBUF, ring_body, 0)


@functools.partial(jax.jit, static_argnames=())
def kernel(x, pe):
    b, h = x.shape
    n = b * h
    assert n % (NUM_WORKERS * GROUP) == 0
    n_groups = n // (NUM_WORKERS * GROUP)
    x2d = x.reshape(n // GROUP, GROUP)

    mesh = plsc.VectorSubcoreMesh(core_axis_name="c", subcore_axis_name="s")
    run = pl.kernel(
        functools.partial(_body, n_groups),
        mesh=mesh,
        out_type=jax.ShapeDtypeStruct((n, D_MODEL), jnp.float32),
        scratch_types=[
            pltpu.VMEM_SHARED((MAX_LEN, D_MODEL), jnp.float32),
            pltpu.VMEM((n_groups, GROUP), jnp.int32),
            [pltpu.VMEM((GROUP, D_MODEL), jnp.float32) for _ in range(NBUF)],
            pltpu.SemaphoreType.DMA,
            [pltpu.SemaphoreType.DMA for _ in range(NBUF)],
        ],
    )
    out = run(pe, x2d)
    return out.reshape(b, h, D_MODEL)
